# baseline (device time: 48090 ns/iter reference)
import jax
import jax.numpy as jnp
from jax import lax
from jax.experimental import pallas as pl
from jax.experimental.pallas import tpu as pltpu

N_DEV = 4


def kernel(x, w_mat, scale_x, scale_w):
    m_per, k = x.shape
    _, n = w_mat.shape
    n_per = n // N_DEV

    def body(x_ref, w_ref, sx_ref, sw_ref, out_ref,
             xv, wv, ybuf, rbuf, obuf, in_sems, out_sems,
             send_sems, recv_sems):
        my = lax.axis_index("i")

        xcp = pltpu.make_async_copy(x_ref, xv, in_sems.at[N_DEV])
        xcp.start()
        wcps = []
        for j in range(N_DEV):
            p = (my + 1 + j) % N_DEV
            wcp = pltpu.make_async_copy(
                w_ref.at[:, pl.ds(p * n_per, n_per)], wv.at[j],
                in_sems.at[j])
            wcp.start()
            wcps.append(wcp)

        barrier_sem = pltpu.get_barrier_semaphore()
        for d in range(1, N_DEV):
            pl.semaphore_signal(
                barrier_sem,
                inc=1,
                device_id=((my + d) % N_DEV,),
                device_id_type=pl.DeviceIdType.MESH,
            )
        pl.semaphore_wait(barrier_sem, N_DEV - 1)

        scale = sx_ref[0] * sw_ref[0]
        xcp.wait()

        for d in range(1, N_DEV):
            j = d - 1
            wcps[j].wait()
            acc = jnp.dot(xv[:, :], wv[j, :, :],
                          preferred_element_type=jnp.int32)
            ybuf[j, :, :] = ((acc + 1024) >> 11).astype(jnp.int16)
            rdma = pltpu.make_async_remote_copy(
                src_ref=ybuf.at[j],
                dst_ref=rbuf.at[j],
                send_sem=send_sems.at[j],
                recv_sem=recv_sems.at[j],
                device_id=((my + d) % N_DEV,),
                device_id_type=pl.DeviceIdType.MESH,
            )
            rdma.start()

        wcps[N_DEV - 1].wait()
        acc = jnp.dot(xv[:, :], wv[N_DEV - 1, :, :],
                      preferred_element_type=jnp.int32)
        obuf[N_DEV - 1, :, :] = acc.astype(jnp.float32) * scale
        ocps = [None] * N_DEV
        ocps[N_DEV - 1] = pltpu.make_async_copy(
            obuf.at[N_DEV - 1],
            out_ref.at[pl.ds(my * m_per, m_per), :],
            out_sems.at[N_DEV - 1])
        ocps[N_DEV - 1].start()

        for d in range(1, N_DEV):
            j = d - 1
            src = (my - d) % N_DEV
            waiter = pltpu.make_async_remote_copy(
                src_ref=ybuf.at[j],
                dst_ref=rbuf.at[j],
                send_sem=send_sems.at[j],
                recv_sem=recv_sems.at[j],
                device_id=((my + d) % N_DEV,),
                device_id_type=pl.DeviceIdType.MESH,
            )
            waiter.wait_recv()
            obuf[j, :, :] = rbuf[j, :, :].astype(jnp.float32) * (2048.0 * scale)
            ocps[j] = pltpu.make_async_copy(
                obuf.at[j],
                out_ref.at[pl.ds(src * m_per, m_per), :],
                out_sems.at[j])
            ocps[j].start()
            waiter.wait_send()

        for j in range(N_DEV):
            ocps[j].wait()

    return pl.pallas_call(
        body,
        out_shape=jax.ShapeDtypeStruct((N_DEV * m_per, n_per), jnp.float32),
        in_specs=[
            pl.BlockSpec(memory_space=pl.ANY),
            pl.BlockSpec(memory_space=pl.ANY),
            pl.BlockSpec(memory_space=pltpu.VMEM),
            pl.BlockSpec(memory_space=pltpu.VMEM),
        ],
        out_specs=pl.BlockSpec(memory_space=pl.ANY),
        scratch_shapes=[
            pltpu.VMEM((m_per, k), jnp.int8),
            pltpu.VMEM((N_DEV, k, n_per), jnp.int8),
            pltpu.VMEM((N_DEV - 1, m_per, n_per), jnp.int16),
            pltpu.VMEM((N_DEV - 1, m_per, n_per), jnp.int16),
            pltpu.VMEM((N_DEV, m_per, n_per), jnp.float32),
            pltpu.SemaphoreType.DMA((N_DEV + 1,)),
            pltpu.SemaphoreType.DMA((N_DEV,)),
            pltpu.SemaphoreType.DMA((N_DEV - 1,)),
            pltpu.SemaphoreType.DMA((N_DEV - 1,)),
        ],
        compiler_params=pltpu.CompilerParams(collective_id=0),
    )(x, w_mat, scale_x, scale_w)
